# baseline (speedup 1.0000x reference)
import jax
import jax.numpy as jnp
from jax import lax
from jax.experimental import pallas as pl
from jax.experimental.pallas import tpu as pltpu
from jax.experimental.pallas import tpu_sc as plsc

_C = 2048
_K = 2048
_COEFFICIENT = 0.001
_Q = 15.0
_STD_DEV_NUM = 2.0

_C_SC = 512
_C_TC = _C - _C_SC

_TC_BLK = 256


def _tc_body(w_ref, s_ref, out_ref):
    i = pl.program_id(0)
    w = w_ref[...]
    s = s_ref[...]
    k = w.shape[1]
    mean = jnp.mean(w, axis=1, keepdims=True)
    var = jnp.sum((w - mean) ** 2, axis=1, keepdims=True) / (k - 1)
    std = jnp.sqrt(var)
    lower = mean - _STD_DEV_NUM * std
    step = (2.0 * _STD_DEV_NUM / _Q) * std
    x = (w - lower) / step
    idx = jnp.floor(jnp.clip(x, 0.0, _Q - 1.0))
    target = s * (idx - 7.0)
    partial = jnp.sum(jnp.abs(w - target)).reshape(1, 1)

    @pl.when(i == 0)
    def _init():
        out_ref[...] = jnp.zeros_like(out_ref)

    out_ref[...] += partial


_L = 16
_NC = 2
_NS = 16
_NW = _NC * _NS
_WROWS = _C_SC // _NW
_HROWS = _WROWS // 2
_UNROLL = 8


def _vrsqrt(v16):
    i = lax.bitcast_convert_type(v16, jnp.int32)
    i = jnp.int32(0x5F3759DF) - (i >> 1)
    y = lax.bitcast_convert_type(i, jnp.float32)
    for _ in range(4):
        y = y * (1.5 - 0.5 * v16 * y * y)
    return y


def _sc_body(w_hbm, s_hbm, out_hbm, bufa, bufb, sbuf, accbuf, sema, semb):
    cid = lax.axis_index("c")
    sid = lax.axis_index("s")
    wid = sid * _NC + cid
    base = _C_TC + wid * _WROWS

    da = pltpu.async_copy(w_hbm.at[pl.ds(base, _HROWS), :], bufa, sema)
    db = pltpu.async_copy(w_hbm.at[pl.ds(base + _HROWS, _HROWS), :], bufb, semb)
    pltpu.sync_copy(s_hbm.at[pl.ds(base, _WROWS), :], sbuf)

    nchunk = _K // (_L * _UNROLL)

    def half_sweep(buf, scale_off, acc_in):
        def row_body(r, acc_row):
            def p1(j, c):
                s, ss = c
                o = j * (_L * _UNROLL)
                for u in range(_UNROLL):
                    v = buf[r, pl.ds(o + u * _L, _L)]
                    s = s + v
                    ss = ss + v * v
                return s, ss

            z = jnp.zeros((_L,), jnp.float32)
            s, ss = lax.fori_loop(0, nchunk, p1, (z, z))
            tot = plsc.cumsum(s)[_L - 1]
            tot2 = plsc.cumsum(ss)[_L - 1]
            mean = tot * (1.0 / _K)
            var = (tot2 - tot * mean) * (1.0 / (_K - 1))
            rstd = _vrsqrt(jnp.full((_L,), var, jnp.float32))
            a = 3.75 * rstd
            cst = 7.5 - mean * a
            ridx = jnp.full((_L,), scale_off + r, jnp.int32)
            sc = plsc.load_gather(sbuf, [ridx, jnp.zeros((_L,), jnp.int32)])
            sc7 = sc * 7.0

            def p2(j, acc):
                o = j * (_L * _UNROLL)
                for u in range(_UNROLL):
                    v = buf[r, pl.ds(o + u * _L, _L)]
                    x = v * a + cst
                    x = jnp.minimum(jnp.maximum(x, 0.0), 14.0)
                    idxf = x.astype(jnp.int32).astype(jnp.float32)
                    acc = acc + jnp.abs(v - (sc * idxf - sc7))
                return acc

            return lax.fori_loop(0, nchunk, p2, acc_row)

        return lax.fori_loop(0, _HROWS, row_body, acc_in)

    z16 = jnp.zeros((_L,), jnp.float32)
    da.wait()
    acc = half_sweep(bufa, 0, z16)
    db.wait()
    acc = half_sweep(bufb, _HROWS, acc)

    accbuf[...] = acc
    pltpu.sync_copy(accbuf, out_hbm.at[wid])


def kernel(weight, scale):
    sc_partials = pl.kernel(
        _sc_body,
        out_type=jax.ShapeDtypeStruct((_NW, _L), jnp.float32),
        mesh=plsc.VectorSubcoreMesh(core_axis_name="c", subcore_axis_name="s"),
        compiler_params=pltpu.CompilerParams(needs_layout_passes=False),
        scratch_types=[
            pltpu.VMEM((_HROWS, _K), jnp.float32),
            pltpu.VMEM((_HROWS, _K), jnp.float32),
            pltpu.VMEM((_WROWS, 1), jnp.float32),
            pltpu.VMEM((_L,), jnp.float32),
            pltpu.SemaphoreType.DMA,
            pltpu.SemaphoreType.DMA,
        ],
    )(weight, scale)

    tc_partial = pl.pallas_call(
        _tc_body,
        grid=(_C_TC // _TC_BLK,),
        in_specs=[
            pl.BlockSpec((_TC_BLK, _K), lambda i: (i, 0)),
            pl.BlockSpec((_TC_BLK, 1), lambda i: (i, 0)),
        ],
        out_specs=pl.BlockSpec((1, 1), lambda i: (0, 0)),
        out_shape=jax.ShapeDtypeStruct((1, 1), jnp.float32),
    )(weight, scale)

    return (tc_partial[0, 0] + jnp.sum(sc_partials)) * _COEFFICIENT


# speedup vs baseline: 1.0182x; 1.0182x over previous
import jax
import jax.numpy as jnp
from jax import lax
from jax.experimental import pallas as pl
from jax.experimental.pallas import tpu as pltpu
from jax.experimental.pallas import tpu_sc as plsc

_C = 2048
_K = 2048
_COEFFICIENT = 0.001
_Q = 15.0
_STD_DEV_NUM = 2.0

_C_SC = 512
_C_TC = _C - _C_SC

_TC_BLK = 256


def _tc_body(w_ref, s_ref, out_ref):
    i = pl.program_id(0)
    w = w_ref[...]
    s = s_ref[...]
    k = w.shape[1]
    mean = jnp.mean(w, axis=1, keepdims=True)
    var = jnp.sum((w - mean) ** 2, axis=1, keepdims=True) / (k - 1)
    std = jnp.sqrt(var)
    lower = mean - _STD_DEV_NUM * std
    step = (2.0 * _STD_DEV_NUM / _Q) * std
    x = (w - lower) / step
    idx = jnp.floor(jnp.clip(x, 0.0, _Q - 1.0))
    target = s * (idx - 7.0)
    partial = jnp.sum(jnp.abs(w - target)).reshape(1, 1)

    @pl.when(i == 0)
    def _init():
        out_ref[...] = jnp.zeros_like(out_ref)

    out_ref[...] += partial


_L = 16
_NC = 2
_NS = 16
_NW = _NC * _NS
_WROWS = _C_SC // _NW
_HROWS = _WROWS // 2
_UNROLL = 8


def _vrsqrt(v16):
    i = lax.bitcast_convert_type(v16, jnp.int32)
    i = jnp.int32(0x5F3759DF) - (i >> 1)
    y = lax.bitcast_convert_type(i, jnp.float32)
    for _ in range(4):
        y = y * (1.5 - 0.5 * v16 * y * y)
    return y


def _sc_body(w_hbm, s_hbm, out_hbm, bufa, bufb, sbuf, accbuf, sema, semb):
    cid = lax.axis_index("c")
    sid = lax.axis_index("s")
    wid = sid * _NC + cid
    base = _C_TC + wid * _WROWS

    da = pltpu.async_copy(w_hbm.at[pl.ds(base, _HROWS), :], bufa, sema)
    db = pltpu.async_copy(w_hbm.at[pl.ds(base + _HROWS, _HROWS), :], bufb, semb)
    pltpu.sync_copy(s_hbm.at[pl.ds(base, _WROWS)], sbuf.at[pl.ds(0, _WROWS)])

    nchunk = _K // (_L * _UNROLL)

    def half_sweep(buf, scale_off, acc_in):
        def row_body(r, acc_row):
            def p1(j, c):
                s, ss = c
                o = j * (_L * _UNROLL)
                for u in range(_UNROLL):
                    v = buf[r, pl.ds(o + u * _L, _L)]
                    s = s + v
                    ss = ss + v * v
                return s, ss

            z = jnp.zeros((_L,), jnp.float32)
            s, ss = lax.fori_loop(0, nchunk, p1, (z, z))
            tot = plsc.cumsum(s)[_L - 1]
            tot2 = plsc.cumsum(ss)[_L - 1]
            mean = tot * (1.0 / _K)
            var = (tot2 - tot * mean) * (1.0 / (_K - 1))
            rstd = _vrsqrt(jnp.full((_L,), var, jnp.float32))
            a = 3.75 * rstd
            cst = 7.5 - mean * a
            sc = sbuf[pl.ds(scale_off + r, _L)][0]
            sc7 = sc * 7.0

            def p2(j, acc):
                o = j * (_L * _UNROLL)
                for u in range(_UNROLL):
                    v = buf[r, pl.ds(o + u * _L, _L)]
                    x = v * a + cst
                    x = jnp.minimum(jnp.maximum(x, 0.0), 14.0)
                    idxf = x.astype(jnp.int32).astype(jnp.float32)
                    acc = acc + jnp.abs(v - (sc * idxf - sc7))
                return acc

            return lax.fori_loop(0, nchunk, p2, acc_row)

        return lax.fori_loop(0, _HROWS, row_body, acc_in)

    z16 = jnp.zeros((_L,), jnp.float32)
    da.wait()
    acc = half_sweep(bufa, 0, z16)
    db.wait()
    acc = half_sweep(bufb, _HROWS, acc)

    accbuf[...] = acc
    pltpu.sync_copy(accbuf, out_hbm.at[wid])


def kernel(weight, scale):
    sc_partials = pl.kernel(
        _sc_body,
        out_type=jax.ShapeDtypeStruct((_NW, _L), jnp.float32),
        mesh=plsc.VectorSubcoreMesh(core_axis_name="c", subcore_axis_name="s"),
        compiler_params=pltpu.CompilerParams(needs_layout_passes=False),
        scratch_types=[
            pltpu.VMEM((_HROWS, _K), jnp.float32),
            pltpu.VMEM((_HROWS, _K), jnp.float32),
            pltpu.VMEM((_WROWS + _L,), jnp.float32),
            pltpu.VMEM((_L,), jnp.float32),
            pltpu.SemaphoreType.DMA,
            pltpu.SemaphoreType.DMA,
        ],
    )(weight, scale.reshape(_C))

    tc_partial = pl.pallas_call(
        _tc_body,
        grid=(_C_TC // _TC_BLK,),
        in_specs=[
            pl.BlockSpec((_TC_BLK, _K), lambda i: (i, 0)),
            pl.BlockSpec((_TC_BLK, 1), lambda i: (i, 0)),
        ],
        out_specs=pl.BlockSpec((1, 1), lambda i: (0, 0)),
        out_shape=jax.ShapeDtypeStruct((1, 1), jnp.float32),
    )(weight, scale)

    return (tc_partial[0, 0] + jnp.sum(sc_partials)) * _COEFFICIENT


# speedup vs baseline: 1.0699x; 1.0507x over previous
import jax
import jax.numpy as jnp
from jax import lax
from jax.experimental import pallas as pl
from jax.experimental.pallas import tpu as pltpu
from jax.experimental.pallas import tpu_sc as plsc

_C = 2048
_K = 2048
_COEFFICIENT = 0.001
_Q = 15.0
_STD_DEV_NUM = 2.0

_C_SC = 256
_C_TC = _C - _C_SC

_TC_BLK = 256


def _tc_body(w_ref, s_ref, out_ref):
    i = pl.program_id(0)
    w = w_ref[...]
    s = s_ref[...]
    k = w.shape[1]
    mean = jnp.mean(w, axis=1, keepdims=True)
    var = jnp.sum((w - mean) ** 2, axis=1, keepdims=True) / (k - 1)
    std = jnp.sqrt(var)
    lower = mean - _STD_DEV_NUM * std
    step = (2.0 * _STD_DEV_NUM / _Q) * std
    x = (w - lower) / step
    idx = jnp.floor(jnp.clip(x, 0.0, _Q - 1.0))
    target = s * (idx - 7.0)
    partial = jnp.sum(jnp.abs(w - target)).reshape(1, 1)

    @pl.when(i == 0)
    def _init():
        out_ref[...] = jnp.zeros_like(out_ref)

    out_ref[...] += partial


_L = 16
_NC = 2
_NS = 16
_NW = _NC * _NS
_WROWS = _C_SC // _NW
_HROWS = _WROWS // 2
_UNROLL = 8


def _vrsqrt(v16):
    i = lax.bitcast_convert_type(v16, jnp.int32)
    i = jnp.int32(0x5F3759DF) - (i >> 1)
    y = lax.bitcast_convert_type(i, jnp.float32)
    for _ in range(4):
        y = y * (1.5 - 0.5 * v16 * y * y)
    return y


def _sc_body(w_hbm, s_hbm, out_hbm, bufa, bufb, sbuf, accbuf, sema, semb):
    cid = lax.axis_index("c")
    sid = lax.axis_index("s")
    wid = sid * _NC + cid
    base = _C_TC + wid * _WROWS

    da = pltpu.async_copy(w_hbm.at[pl.ds(base, _HROWS), :], bufa, sema)
    db = pltpu.async_copy(w_hbm.at[pl.ds(base + _HROWS, _HROWS), :], bufb, semb)
    pltpu.sync_copy(s_hbm.at[pl.ds(base, _WROWS)], sbuf.at[pl.ds(0, _WROWS)])

    nchunk = _K // (_L * _UNROLL)

    def half_sweep(buf, scale_off, acc_in):
        def row_body(r, acc_row):
            def p1(j, c):
                s, ss = c
                o = j * (_L * _UNROLL)
                for u in range(_UNROLL):
                    v = buf[r, pl.ds(o + u * _L, _L)]
                    s = s + v
                    ss = ss + v * v
                return s, ss

            z = jnp.zeros((_L,), jnp.float32)
            s, ss = lax.fori_loop(0, nchunk, p1, (z, z))
            tot = plsc.cumsum(s)[_L - 1]
            tot2 = plsc.cumsum(ss)[_L - 1]
            mean = tot * (1.0 / _K)
            var = (tot2 - tot * mean) * (1.0 / (_K - 1))
            rstd = _vrsqrt(jnp.full((_L,), var, jnp.float32))
            a = 3.75 * rstd
            cst = 7.5 - mean * a
            sc = sbuf[pl.ds(scale_off + r, _L)][0]
            sc7 = sc * 7.0

            def p2(j, acc):
                o = j * (_L * _UNROLL)
                for u in range(_UNROLL):
                    v = buf[r, pl.ds(o + u * _L, _L)]
                    x = v * a + cst
                    x = jnp.minimum(jnp.maximum(x, 0.0), 14.0)
                    idxf = x.astype(jnp.int32).astype(jnp.float32)
                    acc = acc + jnp.abs(v - (sc * idxf - sc7))
                return acc

            return lax.fori_loop(0, nchunk, p2, acc_row)

        return lax.fori_loop(0, _HROWS, row_body, acc_in)

    z16 = jnp.zeros((_L,), jnp.float32)
    da.wait()
    acc = half_sweep(bufa, 0, z16)
    db.wait()
    acc = half_sweep(bufb, _HROWS, acc)

    accbuf[...] = acc
    pltpu.sync_copy(accbuf, out_hbm.at[wid])


def kernel(weight, scale):
    sc_partials = pl.kernel(
        _sc_body,
        out_type=jax.ShapeDtypeStruct((_NW, _L), jnp.float32),
        mesh=plsc.VectorSubcoreMesh(core_axis_name="c", subcore_axis_name="s"),
        compiler_params=pltpu.CompilerParams(needs_layout_passes=False),
        scratch_types=[
            pltpu.VMEM((_HROWS, _K), jnp.float32),
            pltpu.VMEM((_HROWS, _K), jnp.float32),
            pltpu.VMEM((_WROWS + _L,), jnp.float32),
            pltpu.VMEM((_L,), jnp.float32),
            pltpu.SemaphoreType.DMA,
            pltpu.SemaphoreType.DMA,
        ],
    )(weight, scale.reshape(_C))

    tc_partial = pl.pallas_call(
        _tc_body,
        grid=(_C_TC // _TC_BLK,),
        in_specs=[
            pl.BlockSpec((_TC_BLK, _K), lambda i: (i, 0)),
            pl.BlockSpec((_TC_BLK, 1), lambda i: (i, 0)),
        ],
        out_specs=pl.BlockSpec((1, 1), lambda i: (0, 0)),
        out_shape=jax.ShapeDtypeStruct((1, 1), jnp.float32),
    )(weight, scale)

    return (tc_partial[0, 0] + jnp.sum(sc_partials)) * _COEFFICIENT
